# baseline (speedup 1.0000x reference)
import functools

import jax
import jax.numpy as jnp
from jax import lax
from jax.experimental import pallas as pl
from jax.experimental.pallas import tpu as pltpu
from jax.experimental.pallas import tpu_sc as plsc

N = 10000
E = 320000
D_NODE = 128
D_EDGE = 16
H = 64

NC = 2
NS = 16
NW = NC * NS

CHUNK = 128
EPT_CHUNKS = 80
EPT_PAIRS = EPT_CHUNKS // 2
E_PAD = NW * EPT_CHUNKS * CHUNK
SCHUNK = 512
SSUB = SCHUNK // CHUNK
SPT = E_PAD // (NW * SCHUNK)
SPAIRS = SPT // 2
DUMP = 10000
N_T = 10112
ROWS_PER_TILE = N_T // NS

_MESH = plsc.VectorSubcoreMesh(core_axis_name="c", subcore_axis_name="s",
                               num_cores=NC, num_subcores=NS)
_SC_PARAMS = pltpu.CompilerParams(use_tc_tiling_on_sc=False,
                                  needs_layout_passes=False)


def _h0_body(x_ref, nw_ref, nb_ref, out_ref):
    out_ref[...] = jnp.dot(x_ref[...], nw_ref[...],
                           preferred_element_type=jnp.float32) + nb_ref[...]


def _ea_body(eattr_ref, ew_ref, eb2_ref, out_ref):
    out_ref[...] = jnp.dot(eattr_ref[...], ew_ref[...],
                           preferred_element_type=jnp.float32) + eb2_ref[...]


def _mlp_body(g_ref, ea_ref, w1_ref, b1_ref, w2_ref, b2_ref, out_ref):
    m = jnp.concatenate([g_ref[...], ea_ref[...]], axis=1)
    m = jax.nn.relu(jnp.dot(m, w1_ref[...],
                            preferred_element_type=jnp.float32) + b1_ref[...])
    out_ref[...] = jnp.dot(m, w2_ref[...],
                           preferred_element_type=jnp.float32) + b2_ref[...]


def _sum_body(p_ref, out_ref):
    out_ref[...] = p_ref[0] + p_ref[1]


def _final_body(gu_ref, gv_ref, fw1_ref, fb1_ref, fw2_ref, fb2_ref, out_ref):
    er = jnp.concatenate([gu_ref[...], gv_ref[...]], axis=1)
    t = jax.nn.relu(jnp.dot(er, fw1_ref[...],
                            preferred_element_type=jnp.float32) + fb1_ref[...])
    out_ref[...] = jnp.dot(t, fw2_ref[...],
                           preferred_element_type=jnp.float32) + fb2_ref[...]


def _make_gather_body(add_mode):
    def body(h_hbm, src_hbm, dst_hbm, *rest):
        if add_mode:
            (g_hbm, src0, src1, dst0, dst1, gs0, gs1, gd0, gd1, go0, go1,
             six0, six1, sg0, sg1, so0, so1) = rest
        else:
            (gu_hbm, gv_hbm, src0, src1, dst0, dst1, gs0, gs1, gd0, gd1,
             six0, six1, sg0, sg1, so0, so1) = rest
        c = lax.axis_index("c")
        s = lax.axis_index("s")
        wid = c * NS + s
        tbase = wid * EPT_CHUNKS

        srcv = (src0, src1)
        dstv = (dst0, dst1)
        gsv = (gs0, gs1)
        gdv = (gd0, gd1)
        gov = (go0, go1) if add_mode else None
        six = (six0, six1)
        sg = (sg0, sg1)
        so = (so0, so1)

        def issue_idx(cid, b):
            base = (tbase + cid) * CHUNK
            pltpu.async_copy(src_hbm.at[pl.ds(base, CHUNK)], srcv[b], six[b])
            pltpu.async_copy(dst_hbm.at[pl.ds(base, CHUNK)], dstv[b], six[b])

        def wait_idx(b):
            pltpu.make_async_copy(src_hbm.at[pl.ds(0, CHUNK)], srcv[b],
                                  six[b]).wait()
            pltpu.make_async_copy(dst_hbm.at[pl.ds(0, CHUNK)], dstv[b],
                                  six[b]).wait()

        def issue_gather(b):
            pltpu.async_copy(h_hbm.at[srcv[b]], gsv[b], sg[b])
            pltpu.async_copy(h_hbm.at[dstv[b]], gdv[b], sg[b])

        def wait_gather(b):
            pltpu.make_async_copy(h_hbm.at[srcv[b]], gsv[b], sg[b]).wait()
            pltpu.make_async_copy(h_hbm.at[dstv[b]], gdv[b], sg[b]).wait()

        def issue_store(cid, b):
            base = (tbase + cid) * CHUNK
            if add_mode:
                pltpu.async_copy(gov[b], g_hbm.at[pl.ds(base, CHUNK), :], so[b])
            else:
                pltpu.async_copy(gsv[b], gu_hbm.at[pl.ds(base, CHUNK), :], so[b])
                pltpu.async_copy(gdv[b], gv_hbm.at[pl.ds(base, CHUNK), :], so[b])

        def wait_store(b):
            if add_mode:
                pltpu.make_async_copy(gov[b], g_hbm.at[pl.ds(0, CHUNK), :],
                                      so[b]).wait()
            else:
                pltpu.make_async_copy(gsv[b], gu_hbm.at[pl.ds(0, CHUNK), :],
                                      so[b]).wait()
                pltpu.make_async_copy(gdv[b], gv_hbm.at[pl.ds(0, CHUNK), :],
                                      so[b]).wait()

        def compute(b):
            if not add_mode:
                return
            gs_b, gd_b, go_b = gsv[b], gdv[b], gov[b]

            def _erow(e, _):
                for dc in range(H // 16):
                    sl = pl.ds(dc * 16, 16)
                    go_b[e, sl] = gs_b[e, sl] + gd_b[e, sl]
                return 0
            lax.fori_loop(0, CHUNK, _erow, 0, unroll=4)

        issue_idx(0, 0)
        issue_idx(1, 1)
        wait_idx(0)
        issue_gather(0)

        def _pair(ii, _):
            c0 = ii * 2
            wait_gather(0)
            wait_idx(1)
            if add_mode:
                issue_gather(1)

                @pl.when(ii >= 1)
                def _():
                    wait_store(0)
                compute(0)
            else:
                @pl.when(ii >= 1)
                def _():
                    wait_store(1)
                issue_gather(1)
            issue_store(c0, 0)

            @pl.when(ii <= EPT_PAIRS - 2)
            def _():
                issue_idx(c0 + 2, 0)

            wait_gather(1)

            @pl.when(ii <= EPT_PAIRS - 2)
            def _():
                wait_idx(0)
                if not add_mode:
                    wait_store(0)
                issue_gather(0)

            if add_mode:
                @pl.when(ii >= 1)
                def _():
                    wait_store(1)
                compute(1)
            issue_store(c0 + 1, 1)

            @pl.when(ii <= EPT_PAIRS - 2)
            def _():
                issue_idx(c0 + 3, 1)
            return 0
        lax.fori_loop(0, EPT_PAIRS, _pair, 0)
        wait_store(0)
        wait_store(1)

    return body


def _sc_scatter_body(m_hbm, dst2_hbm, zeros_hbm, out_hbm,
                     dst0, dst1, m0, m1,
                     six0, six1, sm0, sm1, ss0, ss1, acc):
    c = lax.axis_index("c")
    s = lax.axis_index("s")
    wid = c * NS + s
    tbase = wid * SPT

    dstv = (dst0, dst1)
    mv = (m0, m1)
    six = (six0, six1)
    sm = (sm0, sm1)
    ss = (ss0, ss1)

    @pl.when(s == 0)
    def _init():
        pltpu.sync_copy(zeros_hbm, acc)
    plsc.subcore_barrier()

    def issue_data(cid, b):
        base = (tbase + cid) * SCHUNK
        rb = (tbase + cid) * SSUB
        pltpu.async_copy(dst2_hbm.at[pl.ds(rb, SSUB), :], dstv[b], six[b])
        pltpu.async_copy(m_hbm.at[pl.ds(base, SCHUNK), :], mv[b], sm[b])

    def wait_data(b):
        pltpu.make_async_copy(dst2_hbm.at[pl.ds(0, SSUB), :], dstv[b],
                              six[b]).wait()
        pltpu.make_async_copy(m_hbm.at[pl.ds(0, SCHUNK), :], mv[b],
                              sm[b]).wait()

    def issue_scatter(b):
        for j in range(SSUB):
            pltpu.async_copy(mv[b].at[pl.ds(j * CHUNK, CHUNK), :],
                             acc.at[dstv[b].at[j]], ss[b], add=True)

    def wait_scatter(b):
        for j in range(SSUB):
            pltpu.make_async_copy(mv[b].at[pl.ds(j * CHUNK, CHUNK), :],
                                  acc.at[dstv[b].at[j]], ss[b]).wait()

    issue_data(0, 0)

    def _pair(ii, _):
        c0 = ii * 2
        wait_data(0)

        @pl.when(ii >= 1)
        def _():
            wait_scatter(1)
        issue_data(c0 + 1, 1)
        issue_scatter(0)

        wait_data(1)

        @pl.when(ii <= SPAIRS - 2)
        def _():
            wait_scatter(0)
            issue_data(c0 + 2, 0)
        issue_scatter(1)
        return 0
    lax.fori_loop(0, SPAIRS, _pair, 0)
    wait_scatter(0)
    wait_scatter(1)
    plsc.subcore_barrier()

    pltpu.sync_copy(acc.at[pl.ds(s * ROWS_PER_TILE, ROWS_PER_TILE), :],
                    out_hbm.at[c, pl.ds(s * ROWS_PER_TILE, ROWS_PER_TILE), :])


_GATHER_SCRATCH = (
    [pltpu.VMEM((CHUNK,), jnp.int32)] * 4
    + [pltpu.VMEM((CHUNK, H), jnp.float32)] * 4
)

_sc_gather_add = functools.partial(
    pl.kernel, _make_gather_body(True),
    out_type=jax.ShapeDtypeStruct((E_PAD, H), jnp.float32),
    mesh=_MESH,
    compiler_params=_SC_PARAMS,
    scratch_types=(
        _GATHER_SCRATCH
        + [pltpu.VMEM((CHUNK, H), jnp.float32)] * 2
        + [pltpu.SemaphoreType.DMA] * 6
    ),
)()

_sc_gather_two = functools.partial(
    pl.kernel, _make_gather_body(False),
    out_type=[jax.ShapeDtypeStruct((E_PAD, H), jnp.float32)] * 2,
    mesh=_MESH,
    compiler_params=_SC_PARAMS,
    scratch_types=(_GATHER_SCRATCH + [pltpu.SemaphoreType.DMA] * 6),
)()

_sc_scatter = functools.partial(
    pl.kernel, _sc_scatter_body,
    out_type=jax.ShapeDtypeStruct((NC, N_T, H), jnp.float32),
    mesh=_MESH,
    compiler_params=_SC_PARAMS,
    scratch_types=(
        [pltpu.VMEM((SSUB, CHUNK), jnp.int32)] * 2
        + [pltpu.VMEM((SCHUNK, H), jnp.float32)] * 2
        + [pltpu.SemaphoreType.DMA] * 6
        + [pltpu.VMEM_SHARED((N_T, H), jnp.float32)]
    ),
)()


def kernel(x, edge_index, edge_attr, node_W, node_b, edge_W, edge_b,
           l0_W1, l0_b1, l0_W2, l0_b2,
           l1_W1, l1_b1, l1_W2, l1_b2,
           l2_W1, l2_b1, l2_W2, l2_b2,
           l3_W1, l3_b1, l3_W2, l3_b2,
           f_W1, f_b1, f_W2, f_b2):
    f32 = jnp.float32
    src = jnp.pad(edge_index[0].astype(jnp.int32), (0, E_PAD - E))
    dst = jnp.pad(edge_index[1].astype(jnp.int32), (0, E_PAD - E),
                  constant_values=DUMP)
    ea_pad = jnp.pad(edge_attr, ((0, E_PAD - E), (0, 0)))
    x_pad = jnp.pad(x, ((0, N_T - N), (0, 0)))
    nb2 = node_b.reshape(1, H)
    eb2 = edge_b.reshape(1, H)
    b1s = [b.reshape(1, H) for b in (l0_b1, l1_b1, l2_b1, l3_b1)]
    b2s = [b.reshape(1, H) for b in (l0_b2, l1_b2, l2_b2, l3_b2)]
    fb1_2 = f_b1.reshape(1, H)
    fb2_2 = f_b2.reshape(1, 1)
    zeros_acc = jnp.zeros((N_T, H), f32)
    dst2 = dst.reshape(E_PAD // CHUNK, CHUNK)

    def full(shape):
        return pl.BlockSpec(shape, lambda i: (0, 0))

    h = pl.pallas_call(
        _h0_body,
        out_shape=jax.ShapeDtypeStruct((N_T, H), f32),
    )(x_pad, node_W, nb2)

    BE = 4096
    egrid = E_PAD // BE
    ea = pl.pallas_call(
        _ea_body,
        grid=(egrid,),
        in_specs=[pl.BlockSpec((BE, D_EDGE), lambda i: (i, 0)),
                  full((D_EDGE, H)), full((1, H))],
        out_specs=pl.BlockSpec((BE, H), lambda i: (i, 0)),
        out_shape=jax.ShapeDtypeStruct((E_PAD, H), f32),
    )(ea_pad, edge_W, eb2)

    w1s = [l0_W1, l1_W1, l2_W1, l3_W1]
    w2s = [l0_W2, l1_W2, l2_W2, l3_W2]
    for l in range(4):
        g = _sc_gather_add(h, src, dst)
        m = pl.pallas_call(
            _mlp_body,
            grid=(egrid,),
            in_specs=[pl.BlockSpec((BE, H), lambda i: (i, 0)),
                      pl.BlockSpec((BE, H), lambda i: (i, 0)),
                      full((2 * H, H)), full((1, H)),
                      full((H, H)), full((1, H))],
            out_specs=pl.BlockSpec((BE, H), lambda i: (i, 0)),
            out_shape=jax.ShapeDtypeStruct((E_PAD, H), f32),
        )(g, ea, w1s[l], b1s[l], w2s[l], b2s[l])
        p = _sc_scatter(m, dst2, zeros_acc)
        h = pl.pallas_call(
            _sum_body,
            out_shape=jax.ShapeDtypeStruct((N_T, H), f32),
        )(p)

    gu, gv = _sc_gather_two(h, src, dst)
    out = pl.pallas_call(
        _final_body,
        grid=(egrid,),
        in_specs=[pl.BlockSpec((BE, H), lambda i: (i, 0)),
                  pl.BlockSpec((BE, H), lambda i: (i, 0)),
                  full((2 * H, H)), full((1, H)),
                  full((H, 1)), full((1, 1))],
        out_specs=pl.BlockSpec((BE, 1), lambda i: (i, 0)),
        out_shape=jax.ShapeDtypeStruct((E_PAD, 1), f32),
    )(gu, gv, f_W1, fb1_2, f_W2, fb2_2)
    return out[:E, 0]


# speedup vs baseline: 1.0326x; 1.0326x over previous
import functools

import jax
import jax.numpy as jnp
from jax import lax
from jax.experimental import pallas as pl
from jax.experimental.pallas import tpu as pltpu
from jax.experimental.pallas import tpu_sc as plsc

N = 10000
E = 320000
D_NODE = 128
D_EDGE = 16
H = 64

NC = 2
NS = 16
NW = NC * NS

CHUNK = 128
EPT_CHUNKS = 80
EPT_PAIRS = EPT_CHUNKS // 2
E_PAD = NW * EPT_CHUNKS * CHUNK
SCHUNK = 512
SSUB = SCHUNK // CHUNK
SPT = E_PAD // (NW * SCHUNK)
SPAIRS = SPT // 2
DUMP = 10000
N_T = 10112
ROWS_PER_TILE = N_T // NS

_MESH = plsc.VectorSubcoreMesh(core_axis_name="c", subcore_axis_name="s",
                               num_cores=NC, num_subcores=NS)
_SC_PARAMS = pltpu.CompilerParams(use_tc_tiling_on_sc=False,
                                  needs_layout_passes=False)


def _h0_body(x_ref, nw_ref, nb_ref, out_ref):
    out_ref[...] = jnp.dot(x_ref[...], nw_ref[...],
                           preferred_element_type=jnp.float32) + nb_ref[...]


def _ea_body(eattr_ref, ew_ref, eb2_ref, out_ref):
    out_ref[...] = jnp.dot(eattr_ref[...], ew_ref[...],
                           preferred_element_type=jnp.float32) + eb2_ref[...]


def _mlp_body(g_ref, ea_ref, w1_ref, b1_ref, w2_ref, b2_ref, out_ref):
    m = jnp.concatenate([g_ref[...], ea_ref[...]], axis=1)
    m = jax.nn.relu(jnp.dot(m, w1_ref[...],
                            preferred_element_type=jnp.float32) + b1_ref[...])
    out_ref[...] = jnp.dot(m, w2_ref[...],
                           preferred_element_type=jnp.float32) + b2_ref[...]


def _sum_body(p_ref, out_ref):
    out_ref[...] = p_ref[0] + p_ref[1]


def _final_body(gu_ref, gv_ref, fw1_ref, fb1_ref, fw2_ref, fb2_ref, out_ref):
    er = jnp.concatenate([gu_ref[...], gv_ref[...]], axis=1)
    t = jax.nn.relu(jnp.dot(er, fw1_ref[...],
                            preferred_element_type=jnp.float32) + fb1_ref[...])
    out_ref[...] = jnp.dot(t, fw2_ref[...],
                           preferred_element_type=jnp.float32) + fb2_ref[...]


def _make_gather_body(add_mode):
    def body(h_hbm, src_hbm, dst_hbm, *rest):
        if add_mode:
            (g_hbm, *sc) = rest
            srcv, dstv, gsv, gdv, gov = (sc[0:4], sc[4:8], sc[8:12],
                                         sc[12:16], sc[16:20])
            six, sg, so = sc[20:24], sc[24:28], sc[28:32]
        else:
            (gu_hbm, gv_hbm, *sc) = rest
            srcv, dstv, gsv, gdv = sc[0:4], sc[4:8], sc[8:12], sc[12:16]
            gov = None
            six, sg, so = sc[16:20], sc[20:24], sc[24:28]
        c = lax.axis_index("c")
        s = lax.axis_index("s")
        wid = c * NS + s
        tbase = wid * EPT_CHUNKS

        def issue_idx(cid, b):
            base = (tbase + cid) * CHUNK
            pltpu.async_copy(src_hbm.at[pl.ds(base, CHUNK)], srcv[b], six[b])
            pltpu.async_copy(dst_hbm.at[pl.ds(base, CHUNK)], dstv[b], six[b])

        def wait_idx(b):
            pltpu.make_async_copy(src_hbm.at[pl.ds(0, CHUNK)], srcv[b],
                                  six[b]).wait()
            pltpu.make_async_copy(dst_hbm.at[pl.ds(0, CHUNK)], dstv[b],
                                  six[b]).wait()

        def issue_gather(b):
            pltpu.async_copy(h_hbm.at[srcv[b]], gsv[b], sg[b])
            pltpu.async_copy(h_hbm.at[dstv[b]], gdv[b], sg[b])

        def wait_gather(b):
            pltpu.make_async_copy(h_hbm.at[srcv[b]], gsv[b], sg[b]).wait()
            pltpu.make_async_copy(h_hbm.at[dstv[b]], gdv[b], sg[b]).wait()

        def issue_store(cid, b):
            base = (tbase + cid) * CHUNK
            if add_mode:
                pltpu.async_copy(gov[b], g_hbm.at[pl.ds(base, CHUNK), :], so[b])
            else:
                pltpu.async_copy(gsv[b], gu_hbm.at[pl.ds(base, CHUNK), :], so[b])
                pltpu.async_copy(gdv[b], gv_hbm.at[pl.ds(base, CHUNK), :], so[b])

        def wait_store(b):
            if add_mode:
                pltpu.make_async_copy(gov[b], g_hbm.at[pl.ds(0, CHUNK), :],
                                      so[b]).wait()
            else:
                pltpu.make_async_copy(gsv[b], gu_hbm.at[pl.ds(0, CHUNK), :],
                                      so[b]).wait()
                pltpu.make_async_copy(gdv[b], gv_hbm.at[pl.ds(0, CHUNK), :],
                                      so[b]).wait()

        def compute(b):
            if not add_mode:
                return
            gs_b, gd_b, go_b = gsv[b], gdv[b], gov[b]

            def _erow(e, _):
                for dc in range(H // 16):
                    sl = pl.ds(dc * 16, 16)
                    go_b[e, sl] = gs_b[e, sl] + gd_b[e, sl]
                return 0
            lax.fori_loop(0, CHUNK, _erow, 0, unroll=4)

        LAST = EPT_CHUNKS - 1
        for b in range(4):
            issue_idx(b, b)
        for b in range(3):
            wait_idx(b)
            issue_gather(b)

        def _quad(ii, _):
            c0 = ii * 4
            for k in range(4):
                cc = c0 + k
                b = k
                nb = (k + 3) % 4
                wait_gather(b)

                @pl.when(cc + 3 <= LAST)
                def _():
                    wait_idx(nb)
                    if not add_mode:
                        @pl.when(cc >= 1)
                        def _():
                            wait_store(nb)
                    issue_gather(nb)

                if add_mode:
                    @pl.when(cc >= 4)
                    def _():
                        wait_store(b)
                    compute(b)
                issue_store(cc, b)

                @pl.when(cc + 4 <= LAST)
                def _():
                    issue_idx(cc + 4, b)
            return 0
        lax.fori_loop(0, EPT_CHUNKS // 4, _quad, 0)
        for b in range(4):
            wait_store(b)

    return body


def _sc_scatter_body(m_hbm, dst2_hbm, zeros_hbm, out_hbm,
                     dst0, dst1, m0, m1,
                     six0, six1, sm0, sm1, ss0, ss1, acc):
    c = lax.axis_index("c")
    s = lax.axis_index("s")
    wid = c * NS + s
    tbase = wid * SPT

    dstv = (dst0, dst1)
    mv = (m0, m1)
    six = (six0, six1)
    sm = (sm0, sm1)
    ss = (ss0, ss1)

    @pl.when(s == 0)
    def _init():
        pltpu.sync_copy(zeros_hbm, acc)
    plsc.subcore_barrier()

    def issue_data(cid, b):
        base = (tbase + cid) * SCHUNK
        rb = (tbase + cid) * SSUB
        pltpu.async_copy(dst2_hbm.at[pl.ds(rb, SSUB), :], dstv[b], six[b])
        pltpu.async_copy(m_hbm.at[pl.ds(base, SCHUNK), :], mv[b], sm[b])

    def wait_data(b):
        pltpu.make_async_copy(dst2_hbm.at[pl.ds(0, SSUB), :], dstv[b],
                              six[b]).wait()
        pltpu.make_async_copy(m_hbm.at[pl.ds(0, SCHUNK), :], mv[b],
                              sm[b]).wait()

    def issue_scatter(b):
        for j in range(SSUB):
            pltpu.async_copy(mv[b].at[pl.ds(j * CHUNK, CHUNK), :],
                             acc.at[dstv[b].at[j]], ss[b], add=True)

    def wait_scatter(b):
        for j in range(SSUB):
            pltpu.make_async_copy(mv[b].at[pl.ds(j * CHUNK, CHUNK), :],
                                  acc.at[dstv[b].at[j]], ss[b]).wait()

    issue_data(0, 0)

    def _pair(ii, _):
        c0 = ii * 2
        wait_data(0)

        @pl.when(ii >= 1)
        def _():
            wait_scatter(1)
        issue_data(c0 + 1, 1)
        issue_scatter(0)

        wait_data(1)

        @pl.when(ii <= SPAIRS - 2)
        def _():
            wait_scatter(0)
            issue_data(c0 + 2, 0)
        issue_scatter(1)
        return 0
    lax.fori_loop(0, SPAIRS, _pair, 0)
    wait_scatter(0)
    wait_scatter(1)
    plsc.subcore_barrier()

    pltpu.sync_copy(acc.at[pl.ds(s * ROWS_PER_TILE, ROWS_PER_TILE), :],
                    out_hbm.at[c, pl.ds(s * ROWS_PER_TILE, ROWS_PER_TILE), :])


_GATHER_SCRATCH = (
    [pltpu.VMEM((CHUNK,), jnp.int32)] * 8
    + [pltpu.VMEM((CHUNK, H), jnp.float32)] * 8
)

_sc_gather_add = functools.partial(
    pl.kernel, _make_gather_body(True),
    out_type=jax.ShapeDtypeStruct((E_PAD, H), jnp.float32),
    mesh=_MESH,
    compiler_params=_SC_PARAMS,
    scratch_types=(
        _GATHER_SCRATCH
        + [pltpu.VMEM((CHUNK, H), jnp.float32)] * 4
        + [pltpu.SemaphoreType.DMA] * 12
    ),
)()

_sc_gather_two = functools.partial(
    pl.kernel, _make_gather_body(False),
    out_type=[jax.ShapeDtypeStruct((E_PAD, H), jnp.float32)] * 2,
    mesh=_MESH,
    compiler_params=_SC_PARAMS,
    scratch_types=(_GATHER_SCRATCH + [pltpu.SemaphoreType.DMA] * 12),
)()

_sc_scatter = functools.partial(
    pl.kernel, _sc_scatter_body,
    out_type=jax.ShapeDtypeStruct((NC, N_T, H), jnp.float32),
    mesh=_MESH,
    compiler_params=_SC_PARAMS,
    scratch_types=(
        [pltpu.VMEM((SSUB, CHUNK), jnp.int32)] * 2
        + [pltpu.VMEM((SCHUNK, H), jnp.float32)] * 2
        + [pltpu.SemaphoreType.DMA] * 6
        + [pltpu.VMEM_SHARED((N_T, H), jnp.float32)]
    ),
)()


def kernel(x, edge_index, edge_attr, node_W, node_b, edge_W, edge_b,
           l0_W1, l0_b1, l0_W2, l0_b2,
           l1_W1, l1_b1, l1_W2, l1_b2,
           l2_W1, l2_b1, l2_W2, l2_b2,
           l3_W1, l3_b1, l3_W2, l3_b2,
           f_W1, f_b1, f_W2, f_b2):
    f32 = jnp.float32
    src = jnp.pad(edge_index[0].astype(jnp.int32), (0, E_PAD - E))
    dst = jnp.pad(edge_index[1].astype(jnp.int32), (0, E_PAD - E),
                  constant_values=DUMP)
    ea_pad = jnp.pad(edge_attr, ((0, E_PAD - E), (0, 0)))
    x_pad = jnp.pad(x, ((0, N_T - N), (0, 0)))
    nb2 = node_b.reshape(1, H)
    eb2 = edge_b.reshape(1, H)
    b1s = [b.reshape(1, H) for b in (l0_b1, l1_b1, l2_b1, l3_b1)]
    b2s = [b.reshape(1, H) for b in (l0_b2, l1_b2, l2_b2, l3_b2)]
    fb1_2 = f_b1.reshape(1, H)
    fb2_2 = f_b2.reshape(1, 1)
    zeros_acc = jnp.zeros((N_T, H), f32)
    dst2 = dst.reshape(E_PAD // CHUNK, CHUNK)

    def full(shape):
        return pl.BlockSpec(shape, lambda i: (0, 0))

    h = pl.pallas_call(
        _h0_body,
        out_shape=jax.ShapeDtypeStruct((N_T, H), f32),
    )(x_pad, node_W, nb2)

    BE = 4096
    egrid = E_PAD // BE
    ea = pl.pallas_call(
        _ea_body,
        grid=(egrid,),
        in_specs=[pl.BlockSpec((BE, D_EDGE), lambda i: (i, 0)),
                  full((D_EDGE, H)), full((1, H))],
        out_specs=pl.BlockSpec((BE, H), lambda i: (i, 0)),
        out_shape=jax.ShapeDtypeStruct((E_PAD, H), f32),
    )(ea_pad, edge_W, eb2)

    w1s = [l0_W1, l1_W1, l2_W1, l3_W1]
    w2s = [l0_W2, l1_W2, l2_W2, l3_W2]
    for l in range(4):
        g = _sc_gather_add(h, src, dst)
        m = pl.pallas_call(
            _mlp_body,
            grid=(egrid,),
            in_specs=[pl.BlockSpec((BE, H), lambda i: (i, 0)),
                      pl.BlockSpec((BE, H), lambda i: (i, 0)),
                      full((2 * H, H)), full((1, H)),
                      full((H, H)), full((1, H))],
            out_specs=pl.BlockSpec((BE, H), lambda i: (i, 0)),
            out_shape=jax.ShapeDtypeStruct((E_PAD, H), f32),
        )(g, ea, w1s[l], b1s[l], w2s[l], b2s[l])
        p = _sc_scatter(m, dst2, zeros_acc)
        h = pl.pallas_call(
            _sum_body,
            out_shape=jax.ShapeDtypeStruct((N_T, H), f32),
        )(p)

    gu, gv = _sc_gather_two(h, src, dst)
    out = pl.pallas_call(
        _final_body,
        grid=(egrid,),
        in_specs=[pl.BlockSpec((BE, H), lambda i: (i, 0)),
                  pl.BlockSpec((BE, H), lambda i: (i, 0)),
                  full((2 * H, H)), full((1, H)),
                  full((H, 1)), full((1, 1))],
        out_specs=pl.BlockSpec((BE, 1), lambda i: (i, 0)),
        out_shape=jax.ShapeDtypeStruct((E_PAD, 1), f32),
    )(gu, gv, f_W1, fb1_2, f_W2, fb2_2)
    return out[:E, 0]


# speedup vs baseline: 1.0340x; 1.0014x over previous
import functools

import jax
import jax.numpy as jnp
from jax import lax
from jax.experimental import pallas as pl
from jax.experimental.pallas import tpu as pltpu
from jax.experimental.pallas import tpu_sc as plsc

N = 10000
E = 320000
D_NODE = 128
D_EDGE = 16
H = 64

NC = 2
NS = 16
NW = NC * NS

CHUNK = 128
EPT_CHUNKS = 80
EPT_PAIRS = EPT_CHUNKS // 2
E_PAD = NW * EPT_CHUNKS * CHUNK
SCHUNK = 256
SSUB = SCHUNK // CHUNK
SPT = E_PAD // (NW * SCHUNK)
SPAIRS = SPT // 2
DUMP = 10000
N_T = 10112
ROWS_PER_TILE = N_T // NS

_MESH = plsc.VectorSubcoreMesh(core_axis_name="c", subcore_axis_name="s",
                               num_cores=NC, num_subcores=NS)
_SC_PARAMS = pltpu.CompilerParams(use_tc_tiling_on_sc=False,
                                  needs_layout_passes=False)


def _h0_body(x_ref, nw_ref, nb_ref, out_ref):
    out_ref[...] = jnp.dot(x_ref[...], nw_ref[...],
                           preferred_element_type=jnp.float32) + nb_ref[...]


def _ea_body(eattr_ref, ew_ref, eb2_ref, out_ref):
    out_ref[...] = jnp.dot(eattr_ref[...], ew_ref[...],
                           preferred_element_type=jnp.float32) + eb2_ref[...]


def _mlp_body(g_ref, ea_ref, w1_ref, b1_ref, w2_ref, b2_ref, out_ref):
    m = jnp.concatenate([g_ref[...], ea_ref[...]], axis=1)
    m = jax.nn.relu(jnp.dot(m, w1_ref[...],
                            preferred_element_type=jnp.float32) + b1_ref[...])
    out_ref[...] = jnp.dot(m, w2_ref[...],
                           preferred_element_type=jnp.float32) + b2_ref[...]


def _sum_body(p_ref, out_ref):
    out_ref[...] = p_ref[0] + p_ref[1]


def _final_body(gu_ref, gv_ref, fw1_ref, fb1_ref, fw2_ref, fb2_ref, out_ref):
    er = jnp.concatenate([gu_ref[...], gv_ref[...]], axis=1)
    t = jax.nn.relu(jnp.dot(er, fw1_ref[...],
                            preferred_element_type=jnp.float32) + fb1_ref[...])
    out_ref[...] = jnp.dot(t, fw2_ref[...],
                           preferred_element_type=jnp.float32) + fb2_ref[...]


def _make_gather_body(add_mode):
    def body(h_hbm, src_hbm, dst_hbm, *rest):
        if add_mode:
            (g_hbm, *sc) = rest
            srcv, dstv, gsv, gdv, gov = (sc[0:4], sc[4:8], sc[8:12],
                                         sc[12:16], sc[16:20])
            six, sg, so = sc[20:24], sc[24:28], sc[28:32]
        else:
            (gu_hbm, gv_hbm, *sc) = rest
            srcv, dstv, gsv, gdv = sc[0:4], sc[4:8], sc[8:12], sc[12:16]
            gov = None
            six, sg, so = sc[16:20], sc[20:24], sc[24:28]
        c = lax.axis_index("c")
        s = lax.axis_index("s")
        wid = c * NS + s
        tbase = wid * EPT_CHUNKS

        def issue_idx(cid, b):
            base = (tbase + cid) * CHUNK
            pltpu.async_copy(src_hbm.at[pl.ds(base, CHUNK)], srcv[b], six[b])
            pltpu.async_copy(dst_hbm.at[pl.ds(base, CHUNK)], dstv[b], six[b])

        def wait_idx(b):
            pltpu.make_async_copy(src_hbm.at[pl.ds(0, CHUNK)], srcv[b],
                                  six[b]).wait()
            pltpu.make_async_copy(dst_hbm.at[pl.ds(0, CHUNK)], dstv[b],
                                  six[b]).wait()

        def issue_gather(b):
            pltpu.async_copy(h_hbm.at[srcv[b]], gsv[b], sg[b])
            pltpu.async_copy(h_hbm.at[dstv[b]], gdv[b], sg[b])

        def wait_gather(b):
            pltpu.make_async_copy(h_hbm.at[srcv[b]], gsv[b], sg[b]).wait()
            pltpu.make_async_copy(h_hbm.at[dstv[b]], gdv[b], sg[b]).wait()

        def issue_store(cid, b):
            base = (tbase + cid) * CHUNK
            if add_mode:
                pltpu.async_copy(gov[b], g_hbm.at[pl.ds(base, CHUNK), :], so[b])
            else:
                pltpu.async_copy(gsv[b], gu_hbm.at[pl.ds(base, CHUNK), :], so[b])
                pltpu.async_copy(gdv[b], gv_hbm.at[pl.ds(base, CHUNK), :], so[b])

        def wait_store(b):
            if add_mode:
                pltpu.make_async_copy(gov[b], g_hbm.at[pl.ds(0, CHUNK), :],
                                      so[b]).wait()
            else:
                pltpu.make_async_copy(gsv[b], gu_hbm.at[pl.ds(0, CHUNK), :],
                                      so[b]).wait()
                pltpu.make_async_copy(gdv[b], gv_hbm.at[pl.ds(0, CHUNK), :],
                                      so[b]).wait()

        def compute(b):
            if not add_mode:
                return
            gs_b, gd_b, go_b = gsv[b], gdv[b], gov[b]

            def _erow(e, _):
                for dc in range(H // 16):
                    sl = pl.ds(dc * 16, 16)
                    go_b[e, sl] = gs_b[e, sl] + gd_b[e, sl]
                return 0
            lax.fori_loop(0, CHUNK, _erow, 0, unroll=4)

        LAST = EPT_CHUNKS - 1
        for b in range(4):
            issue_idx(b, b)
        for b in range(3):
            wait_idx(b)
            issue_gather(b)

        def _quad(ii, _):
            c0 = ii * 4
            for k in range(4):
                cc = c0 + k
                b = k
                nb = (k + 3) % 4
                wait_gather(b)

                @pl.when(cc + 3 <= LAST)
                def _():
                    wait_idx(nb)
                    if not add_mode:
                        @pl.when(cc >= 1)
                        def _():
                            wait_store(nb)
                    issue_gather(nb)

                if add_mode:
                    @pl.when(cc >= 4)
                    def _():
                        wait_store(b)
                    compute(b)
                issue_store(cc, b)

                @pl.when(cc + 4 <= LAST)
                def _():
                    issue_idx(cc + 4, b)
            return 0
        lax.fori_loop(0, EPT_CHUNKS // 4, _quad, 0)
        for b in range(4):
            wait_store(b)

    return body


def _sc_scatter_body(m_hbm, dst2_hbm, zeros_hbm, out_hbm, *sc):
    dstv, mv = sc[0:4], sc[4:8]
    six, sm, ss = sc[8:12], sc[12:16], sc[16:20]
    acc = sc[20]
    c = lax.axis_index("c")
    s = lax.axis_index("s")
    wid = c * NS + s
    tbase = wid * SPT

    @pl.when(s == 0)
    def _init():
        pltpu.sync_copy(zeros_hbm, acc)
    plsc.subcore_barrier()

    def issue_data(cid, b):
        base = (tbase + cid) * SCHUNK
        rb = (tbase + cid) * SSUB
        pltpu.async_copy(dst2_hbm.at[pl.ds(rb, SSUB), :], dstv[b], six[b])
        pltpu.async_copy(m_hbm.at[pl.ds(base, SCHUNK), :], mv[b], sm[b])

    def wait_data(b):
        pltpu.make_async_copy(dst2_hbm.at[pl.ds(0, SSUB), :], dstv[b],
                              six[b]).wait()
        pltpu.make_async_copy(m_hbm.at[pl.ds(0, SCHUNK), :], mv[b],
                              sm[b]).wait()

    def issue_scatter(b):
        for j in range(SSUB):
            pltpu.async_copy(mv[b].at[pl.ds(j * CHUNK, CHUNK), :],
                             acc.at[dstv[b].at[j]], ss[b], add=True)

    def wait_scatter(b):
        for j in range(SSUB):
            pltpu.make_async_copy(mv[b].at[pl.ds(j * CHUNK, CHUNK), :],
                                  acc.at[dstv[b].at[j]], ss[b]).wait()

    LAST = SPT - 1
    for b in range(3):
        issue_data(b, b)

    def _quad(ii, _):
        c0 = ii * 4
        for k in range(4):
            cc = c0 + k
            b = k
            nb = (k + 3) % 4
            wait_data(b)

            @pl.when(cc + 3 <= LAST)
            def _():
                @pl.when(cc >= 1)
                def _():
                    wait_scatter(nb)
                issue_data(cc + 3, nb)
            issue_scatter(b)
        return 0
    lax.fori_loop(0, SPT // 4, _quad, 0)
    for b in range(4):
        wait_scatter(b)
    plsc.subcore_barrier()

    pltpu.sync_copy(acc.at[pl.ds(s * ROWS_PER_TILE, ROWS_PER_TILE), :],
                    out_hbm.at[c, pl.ds(s * ROWS_PER_TILE, ROWS_PER_TILE), :])


_GATHER_SCRATCH = (
    [pltpu.VMEM((CHUNK,), jnp.int32)] * 8
    + [pltpu.VMEM((CHUNK, H), jnp.float32)] * 8
)

_sc_gather_add = functools.partial(
    pl.kernel, _make_gather_body(True),
    out_type=jax.ShapeDtypeStruct((E_PAD, H), jnp.float32),
    mesh=_MESH,
    compiler_params=_SC_PARAMS,
    scratch_types=(
        _GATHER_SCRATCH
        + [pltpu.VMEM((CHUNK, H), jnp.float32)] * 4
        + [pltpu.SemaphoreType.DMA] * 12
    ),
)()

_sc_gather_two = functools.partial(
    pl.kernel, _make_gather_body(False),
    out_type=[jax.ShapeDtypeStruct((E_PAD, H), jnp.float32)] * 2,
    mesh=_MESH,
    compiler_params=_SC_PARAMS,
    scratch_types=(_GATHER_SCRATCH + [pltpu.SemaphoreType.DMA] * 12),
)()

_sc_scatter = functools.partial(
    pl.kernel, _sc_scatter_body,
    out_type=jax.ShapeDtypeStruct((NC, N_T, H), jnp.float32),
    mesh=_MESH,
    compiler_params=_SC_PARAMS,
    scratch_types=(
        [pltpu.VMEM((SSUB, CHUNK), jnp.int32)] * 4
        + [pltpu.VMEM((SCHUNK, H), jnp.float32)] * 4
        + [pltpu.SemaphoreType.DMA] * 12
        + [pltpu.VMEM_SHARED((N_T, H), jnp.float32)]
    ),
)()


def kernel(x, edge_index, edge_attr, node_W, node_b, edge_W, edge_b,
           l0_W1, l0_b1, l0_W2, l0_b2,
           l1_W1, l1_b1, l1_W2, l1_b2,
           l2_W1, l2_b1, l2_W2, l2_b2,
           l3_W1, l3_b1, l3_W2, l3_b2,
           f_W1, f_b1, f_W2, f_b2):
    f32 = jnp.float32
    src = jnp.pad(edge_index[0].astype(jnp.int32), (0, E_PAD - E))
    dst = jnp.pad(edge_index[1].astype(jnp.int32), (0, E_PAD - E),
                  constant_values=DUMP)
    ea_pad = jnp.pad(edge_attr, ((0, E_PAD - E), (0, 0)))
    x_pad = jnp.pad(x, ((0, N_T - N), (0, 0)))
    nb2 = node_b.reshape(1, H)
    eb2 = edge_b.reshape(1, H)
    b1s = [b.reshape(1, H) for b in (l0_b1, l1_b1, l2_b1, l3_b1)]
    b2s = [b.reshape(1, H) for b in (l0_b2, l1_b2, l2_b2, l3_b2)]
    fb1_2 = f_b1.reshape(1, H)
    fb2_2 = f_b2.reshape(1, 1)
    zeros_acc = jnp.zeros((N_T, H), f32)
    dst2 = dst.reshape(E_PAD // CHUNK, CHUNK)

    def full(shape):
        return pl.BlockSpec(shape, lambda i: (0, 0))

    h = pl.pallas_call(
        _h0_body,
        out_shape=jax.ShapeDtypeStruct((N_T, H), f32),
    )(x_pad, node_W, nb2)

    BE = 4096
    egrid = E_PAD // BE
    ea = pl.pallas_call(
        _ea_body,
        grid=(egrid,),
        in_specs=[pl.BlockSpec((BE, D_EDGE), lambda i: (i, 0)),
                  full((D_EDGE, H)), full((1, H))],
        out_specs=pl.BlockSpec((BE, H), lambda i: (i, 0)),
        out_shape=jax.ShapeDtypeStruct((E_PAD, H), f32),
    )(ea_pad, edge_W, eb2)

    w1s = [l0_W1, l1_W1, l2_W1, l3_W1]
    w2s = [l0_W2, l1_W2, l2_W2, l3_W2]
    for l in range(4):
        g = _sc_gather_add(h, src, dst)
        m = pl.pallas_call(
            _mlp_body,
            grid=(egrid,),
            in_specs=[pl.BlockSpec((BE, H), lambda i: (i, 0)),
                      pl.BlockSpec((BE, H), lambda i: (i, 0)),
                      full((2 * H, H)), full((1, H)),
                      full((H, H)), full((1, H))],
            out_specs=pl.BlockSpec((BE, H), lambda i: (i, 0)),
            out_shape=jax.ShapeDtypeStruct((E_PAD, H), f32),
        )(g, ea, w1s[l], b1s[l], w2s[l], b2s[l])
        p = _sc_scatter(m, dst2, zeros_acc)
        h = pl.pallas_call(
            _sum_body,
            out_shape=jax.ShapeDtypeStruct((N_T, H), f32),
        )(p)

    gu, gv = _sc_gather_two(h, src, dst)
    out = pl.pallas_call(
        _final_body,
        grid=(egrid,),
        in_specs=[pl.BlockSpec((BE, H), lambda i: (i, 0)),
                  pl.BlockSpec((BE, H), lambda i: (i, 0)),
                  full((2 * H, H)), full((1, H)),
                  full((H, 1)), full((1, 1))],
        out_specs=pl.BlockSpec((BE, 1), lambda i: (i, 0)),
        out_shape=jax.ShapeDtypeStruct((E_PAD, 1), f32),
    )(gu, gv, f_W1, fb1_2, f_W2, fb2_2)
    return out[:E, 0]
